# baseline (speedup 1.0000x reference)
import functools

import jax
import jax.numpy as jnp
from jax import lax
from jax.experimental import pallas as pl
from jax.experimental.pallas import tpu as pltpu
from jax.experimental.pallas import tpu_sc as plsc

NC = 2
NS = 16
L = 16
CHUNK = 64
STAGE = 16


def _round_up(v, m):
    return (v + m - 1) // m * m


def _mesh():
    return plsc.VectorSubcoreMesh(core_axis_name="c", subcore_axis_name="s")


@functools.lru_cache(maxsize=None)
def _make_deg(nch, NH):
    @functools.partial(
        pl.kernel,
        out_type=jax.ShapeDtypeStruct((NC * NS, NH), jnp.float32),
        mesh=_mesh(),
        compiler_params=pltpu.CompilerParams(needs_layout_passes=False),
        scratch_types=[
            pltpu.VMEM((nch, CHUNK), jnp.int32),
            pltpu.VMEM((NH,), jnp.float32),
        ],
    )
    def deg_k(dst_hbm, degp_hbm, didx, hist):
        c = lax.axis_index("c")
        s = lax.axis_index("s")
        w = s * NC + c

        def fill_z(k, carry):
            hist[pl.ds(k * L, L)] = jnp.zeros((L,), jnp.float32)
            return carry

        lax.fori_loop(0, NH // L, fill_z, 0)

        pltpu.sync_copy(dst_hbm.at[w], didx)
        ones = jnp.ones((L,), jnp.float32)

        def chunk_body(j, carry):
            def lane_grp(k, carry2):
                idx = didx[j, pl.ds(k * L, L)]
                plsc.addupdate_scatter(hist, [idx], ones)
                return carry2

            return lax.fori_loop(0, CHUNK // L, lane_grp, carry)

        lax.fori_loop(0, nch, chunk_body, 0)
        pltpu.sync_copy(hist, degp_hbm.at[w])

    return deg_k


FPT = 8
GR = 16
CH = 128


@functools.lru_cache(maxsize=None)
def _make_agg(N, D, nch, NP):
    @functools.partial(
        pl.kernel,
        out_type=jax.ShapeDtypeStruct((NC * NS, NP, FPT), jnp.float32),
        mesh=_mesh(),
        compiler_params=pltpu.CompilerParams(
            needs_layout_passes=False, use_tc_tiling_on_sc=False),
        scratch_types=[
            pltpu.VMEM((STAGE, CH), jnp.int32),
            pltpu.VMEM((STAGE, CH), jnp.int32),
            pltpu.VMEM((CH, GR), jnp.float32),
            pltpu.VMEM((CH, GR), jnp.float32),
            pltpu.VMEM((CH, GR), jnp.float32),
            pltpu.VMEM((CH, GR), jnp.float32),
            pltpu.VMEM((NP, FPT), jnp.float32),
            pltpu.SemaphoreType.DMA,
            pltpu.SemaphoreType.DMA,
            pltpu.SemaphoreType.DMA,
            pltpu.SemaphoreType.DMA,
        ],
    )
    def agg_k(src_hbm, dst_hbm, h2g_hbm, zrows_hbm, acc_hbm, sidx, didx,
              rows0, rows1, rows2, rows3, acc, sem0, sem1, sem2, sem3):
        c = lax.axis_index("c")
        s = lax.axis_index("s")
        t = s * NC + c
        half = t // 2
        off = (t % 2) * FPT

        pltpu.sync_copy(zrows_hbm, acc)

        iota = lax.iota(jnp.int32, L)

        def do_adds(rows, j):
            @plsc.parallel_loop(0, CH // L, step=1, unroll=4)
            def grp_body(k):
                dst16 = didx[j, pl.ds(k * L, L)]
                eidx = iota + k * L
                for f in range(FPT):
                    vals = plsc.load_gather(
                        rows, [eidx, jnp.full((L,), off + f, jnp.int32)])
                    plsc.addupdate_scatter(
                        acc, [dst16, jnp.full((L,), f, jnp.int32)], vals)

        bufs = [(rows0, sem0), (rows1, sem1), (rows2, sem2), (rows3, sem3)]
        DEPTH = len(bufs)

        def gather(j, rows, sem):
            pltpu.async_copy(h2g_hbm.at[half].at[sidx.at[j]], rows, sem)

        def drain(j, rows, sem):
            pltpu.make_async_copy(h2g_hbm.at[half].at[sidx.at[j]], rows,
                                  sem).wait()

        def stage_body(g, carry):
            pltpu.sync_copy(src_hbm.at[pl.ds(g * STAGE, STAGE)], sidx)
            pltpu.sync_copy(dst_hbm.at[pl.ds(g * STAGE, STAGE)], didx)
            for j in range(DEPTH - 1):
                gather(j, *bufs[j])
            for j in range(STAGE):
                drain(j, *bufs[j % DEPTH])
                if j + DEPTH - 1 < STAGE:
                    gather(j + DEPTH - 1, *bufs[(j + DEPTH - 1) % DEPTH])
                do_adds(bufs[j % DEPTH][0], j)
            return carry

        lax.fori_loop(0, nch // STAGE, stage_body, 0)
        pltpu.sync_copy(acc, acc_hbm.at[t])

    return agg_k


def _tc1_body(x_ref, w_ref, d_ref, h2_ref, h2g_ref):
    h = jnp.dot(x_ref[...], w_ref[...], preferred_element_type=jnp.float32)
    deg = jnp.sum(d_ref[...], axis=1, keepdims=True) + 1.0
    h2 = h * lax.rsqrt(deg)
    h2_ref[...] = h2
    br, d = h2.shape
    h2g_ref[...] = jnp.transpose(h2.reshape(br, d // GR, GR), (1, 0, 2))


def _tc2_body(a_ref, h2_ref, d_ref, b1_ref, w_ref, lb_ref, o_ref):
    deg = jnp.sum(d_ref[...], axis=1, keepdims=True) + 1.0
    dis = lax.rsqrt(deg)
    nt, br, fpt = a_ref.shape
    a = jnp.transpose(a_ref[...], (1, 0, 2)).reshape(br, nt * fpt)
    t = jnp.tanh((a + h2_ref[...]) * dis + b1_ref[...])
    o_ref[...] = (jnp.dot(t, w_ref[...], preferred_element_type=jnp.float32)
                  + lb_ref[...])


@functools.lru_cache(maxsize=None)
def _make_tc1(N, D_in, D_hid, BR):
    grid = (N + BR - 1) // BR
    return pl.pallas_call(
        _tc1_body,
        grid=(grid,),
        in_specs=[
            pl.BlockSpec((BR, D_in), lambda i: (i, 0)),
            pl.BlockSpec((D_in, D_hid), lambda i: (0, 0)),
            pl.BlockSpec((BR, NC * NS), lambda i: (i, 0)),
        ],
        out_specs=[
            pl.BlockSpec((BR, D_hid), lambda i: (i, 0)),
            pl.BlockSpec((D_hid // GR, BR, GR), lambda i: (0, i, 0)),
        ],
        out_shape=[
            jax.ShapeDtypeStruct((N, D_hid), jnp.float32),
            jax.ShapeDtypeStruct((D_hid // GR, N, GR), jnp.float32),
        ],
    )


@functools.lru_cache(maxsize=None)
def _make_tc2(N, D_hid, D_out, BR):
    grid = (N + BR - 1) // BR
    return pl.pallas_call(
        _tc2_body,
        grid=(grid,),
        in_specs=[
            pl.BlockSpec((NC * NS, BR, FPT), lambda i: (0, i, 0)),
            pl.BlockSpec((BR, D_hid), lambda i: (i, 0)),
            pl.BlockSpec((BR, NC * NS), lambda i: (i, 0)),
            pl.BlockSpec((1, D_hid), lambda i: (0, 0)),
            pl.BlockSpec((D_hid, D_out), lambda i: (0, 0)),
            pl.BlockSpec((1, D_out), lambda i: (0, 0)),
        ],
        out_specs=pl.BlockSpec((BR, D_out), lambda i: (i, 0)),
        out_shape=jax.ShapeDtypeStruct((N, D_out), jnp.float32),
    )


def kernel(x, edge_index, W1, b1, lin_W, lin_b):
    N, D_in = x.shape
    D_hid = W1.shape[1]
    D_out = lin_W.shape[1]
    E = edge_index.shape[1]

    src = edge_index[0]
    dst = edge_index[1]

    EP = _round_up(E, max(NC * NS * CHUNK, STAGE * CH))
    pad = EP - E
    if pad:
        src = jnp.concatenate([src, jnp.zeros((pad,), src.dtype)])
        dst = jnp.concatenate([dst, jnp.full((pad,), N, dst.dtype)])

    nch_deg = EP // (NC * NS * CHUNK)
    nch_agg = EP // CH
    NH = _round_up(N + 1, 128)
    NP = _round_up(N + 1, 8)

    dst_deg = dst.reshape(NC * NS, nch_deg, CHUNK)
    src2 = src.reshape(nch_agg, CH)
    dst2 = dst.reshape(nch_agg, CH)

    degp = _make_deg(nch_deg, NH)(dst_deg)
    degt = degp[:, :N].T

    BR = 1000 if N % 1000 == 0 else N
    h2, h2g = _make_tc1(N, D_in, D_hid, BR)(x, W1, degt)
    zrows = jnp.zeros((NP, FPT), jnp.float32)
    accg = _make_agg(N, D_hid, nch_agg, NP)(src2, dst2, h2g, zrows)
    out = _make_tc2(N, D_hid, D_out, BR)(
        accg, h2, degt, b1.reshape(1, D_hid), lin_W,
        lin_b.reshape(1, D_out))
    return out


# speedup vs baseline: 1.0043x; 1.0043x over previous
import functools

import jax
import jax.numpy as jnp
from jax import lax
from jax.experimental import pallas as pl
from jax.experimental.pallas import tpu as pltpu
from jax.experimental.pallas import tpu_sc as plsc

NC = 2
NS = 16
L = 16
CHUNK = 64
STAGE = 16


def _round_up(v, m):
    return (v + m - 1) // m * m


def _mesh():
    return plsc.VectorSubcoreMesh(core_axis_name="c", subcore_axis_name="s")


@functools.lru_cache(maxsize=None)
def _make_deg(nch, NH):
    @functools.partial(
        pl.kernel,
        out_type=jax.ShapeDtypeStruct((NC * NS, NH), jnp.float32),
        mesh=_mesh(),
        compiler_params=pltpu.CompilerParams(needs_layout_passes=False),
        scratch_types=[
            pltpu.VMEM((nch, CHUNK), jnp.int32),
            pltpu.VMEM((NH,), jnp.float32),
        ],
    )
    def deg_k(dst_hbm, degp_hbm, didx, hist):
        c = lax.axis_index("c")
        s = lax.axis_index("s")
        w = s * NC + c

        def fill_z(k, carry):
            hist[pl.ds(k * L, L)] = jnp.zeros((L,), jnp.float32)
            return carry

        lax.fori_loop(0, NH // L, fill_z, 0)

        pltpu.sync_copy(dst_hbm.at[w], didx)
        ones = jnp.ones((L,), jnp.float32)

        def chunk_body(j, carry):
            def lane_grp(k, carry2):
                idx = didx[j, pl.ds(k * L, L)]
                plsc.addupdate_scatter(hist, [idx], ones)
                return carry2

            return lax.fori_loop(0, CHUNK // L, lane_grp, carry)

        lax.fori_loop(0, nch, chunk_body, 0)
        pltpu.sync_copy(hist, degp_hbm.at[w])

    return deg_k


FPT = 8
GR = 16
CH = 128


@functools.lru_cache(maxsize=None)
def _make_agg(N, D, nch, NP):
    @functools.partial(
        pl.kernel,
        out_type=jax.ShapeDtypeStruct((NC * NS, NP, FPT), jnp.float32),
        mesh=_mesh(),
        compiler_params=pltpu.CompilerParams(
            needs_layout_passes=False, use_tc_tiling_on_sc=False),
        scratch_types=[
            pltpu.VMEM((STAGE, CH), jnp.int32),
            pltpu.VMEM((STAGE, CH), jnp.int32),
            pltpu.VMEM((CH, GR), jnp.float32),
            pltpu.VMEM((CH, GR), jnp.float32),
            pltpu.VMEM((CH, GR), jnp.float32),
            pltpu.VMEM((CH, GR), jnp.float32),
            pltpu.VMEM((NP, FPT), jnp.float32),
            pltpu.SemaphoreType.DMA,
            pltpu.SemaphoreType.DMA,
            pltpu.SemaphoreType.DMA,
            pltpu.SemaphoreType.DMA,
        ],
    )
    def agg_k(src_hbm, dst_hbm, h2g_hbm, zrows_hbm, acc_hbm, sidx, didx,
              rows0, rows1, rows2, rows3, acc, sem0, sem1, sem2, sem3):
        c = lax.axis_index("c")
        s = lax.axis_index("s")
        t = s * NC + c
        half = t // 2
        off = (t % 2) * FPT

        pltpu.sync_copy(zrows_hbm, acc)

        iota = lax.iota(jnp.int32, L)

        def do_adds(rows, j):
            @plsc.parallel_loop(0, CH // L, step=1, unroll=2)
            def grp_body(k):
                dst16 = didx[j, pl.ds(k * L, L)]
                eidx = iota + k * L
                for f in range(FPT):
                    vals = plsc.load_gather(
                        rows, [eidx, jnp.full((L,), off + f, jnp.int32)])
                    plsc.addupdate_scatter(
                        acc, [dst16, jnp.full((L,), f, jnp.int32)], vals)

        bufs = [(rows0, sem0), (rows1, sem1), (rows2, sem2), (rows3, sem3)]
        DEPTH = len(bufs)

        def gather(j, rows, sem):
            pltpu.async_copy(h2g_hbm.at[half].at[sidx.at[j]], rows, sem)

        def drain(j, rows, sem):
            pltpu.make_async_copy(h2g_hbm.at[half].at[sidx.at[j]], rows,
                                  sem).wait()

        def stage_body(g, carry):
            pltpu.sync_copy(src_hbm.at[pl.ds(g * STAGE, STAGE)], sidx)
            pltpu.sync_copy(dst_hbm.at[pl.ds(g * STAGE, STAGE)], didx)
            for j in range(DEPTH - 1):
                gather(j, *bufs[j])
            for j in range(STAGE):
                drain(j, *bufs[j % DEPTH])
                if j + DEPTH - 1 < STAGE:
                    gather(j + DEPTH - 1, *bufs[(j + DEPTH - 1) % DEPTH])
                do_adds(bufs[j % DEPTH][0], j)
            return carry

        lax.fori_loop(0, nch // STAGE, stage_body, 0)
        pltpu.sync_copy(acc, acc_hbm.at[t])

    return agg_k


def _tc1_body(x_ref, w_ref, d_ref, h2_ref, h2g_ref):
    h = jnp.dot(x_ref[...], w_ref[...], preferred_element_type=jnp.float32)
    deg = jnp.sum(d_ref[...], axis=1, keepdims=True) + 1.0
    h2 = h * lax.rsqrt(deg)
    h2_ref[...] = h2
    br, d = h2.shape
    h2g_ref[...] = jnp.transpose(h2.reshape(br, d // GR, GR), (1, 0, 2))


def _tc2_body(a_ref, h2_ref, d_ref, b1_ref, w_ref, lb_ref, o_ref):
    deg = jnp.sum(d_ref[...], axis=1, keepdims=True) + 1.0
    dis = lax.rsqrt(deg)
    nt, br, fpt = a_ref.shape
    a = jnp.transpose(a_ref[...], (1, 0, 2)).reshape(br, nt * fpt)
    t = jnp.tanh((a + h2_ref[...]) * dis + b1_ref[...])
    o_ref[...] = (jnp.dot(t, w_ref[...], preferred_element_type=jnp.float32)
                  + lb_ref[...])


@functools.lru_cache(maxsize=None)
def _make_tc1(N, D_in, D_hid, BR):
    grid = (N + BR - 1) // BR
    return pl.pallas_call(
        _tc1_body,
        grid=(grid,),
        in_specs=[
            pl.BlockSpec((BR, D_in), lambda i: (i, 0)),
            pl.BlockSpec((D_in, D_hid), lambda i: (0, 0)),
            pl.BlockSpec((BR, NC * NS), lambda i: (i, 0)),
        ],
        out_specs=[
            pl.BlockSpec((BR, D_hid), lambda i: (i, 0)),
            pl.BlockSpec((D_hid // GR, BR, GR), lambda i: (0, i, 0)),
        ],
        out_shape=[
            jax.ShapeDtypeStruct((N, D_hid), jnp.float32),
            jax.ShapeDtypeStruct((D_hid // GR, N, GR), jnp.float32),
        ],
    )


@functools.lru_cache(maxsize=None)
def _make_tc2(N, D_hid, D_out, BR):
    grid = (N + BR - 1) // BR
    return pl.pallas_call(
        _tc2_body,
        grid=(grid,),
        in_specs=[
            pl.BlockSpec((NC * NS, BR, FPT), lambda i: (0, i, 0)),
            pl.BlockSpec((BR, D_hid), lambda i: (i, 0)),
            pl.BlockSpec((BR, NC * NS), lambda i: (i, 0)),
            pl.BlockSpec((1, D_hid), lambda i: (0, 0)),
            pl.BlockSpec((D_hid, D_out), lambda i: (0, 0)),
            pl.BlockSpec((1, D_out), lambda i: (0, 0)),
        ],
        out_specs=pl.BlockSpec((BR, D_out), lambda i: (i, 0)),
        out_shape=jax.ShapeDtypeStruct((N, D_out), jnp.float32),
    )


def kernel(x, edge_index, W1, b1, lin_W, lin_b):
    N, D_in = x.shape
    D_hid = W1.shape[1]
    D_out = lin_W.shape[1]
    E = edge_index.shape[1]

    src = edge_index[0]
    dst = edge_index[1]

    EP = _round_up(E, max(NC * NS * CHUNK, STAGE * CH))
    pad = EP - E
    if pad:
        src = jnp.concatenate([src, jnp.zeros((pad,), src.dtype)])
        dst = jnp.concatenate([dst, jnp.full((pad,), N, dst.dtype)])

    nch_deg = EP // (NC * NS * CHUNK)
    nch_agg = EP // CH
    NH = _round_up(N + 1, 128)
    NP = _round_up(N + 1, 8)

    dst_deg = dst.reshape(NC * NS, nch_deg, CHUNK)
    src2 = src.reshape(nch_agg, CH)
    dst2 = dst.reshape(nch_agg, CH)

    degp = _make_deg(nch_deg, NH)(dst_deg)
    degt = degp[:, :N].T

    BR = 1000 if N % 1000 == 0 else N
    h2, h2g = _make_tc1(N, D_in, D_hid, BR)(x, W1, degt)
    zrows = jnp.zeros((NP, FPT), jnp.float32)
    accg = _make_agg(N, D_hid, nch_agg, NP)(src2, dst2, h2g, zrows)
    out = _make_tc2(N, D_hid, D_out, BR)(
        accg, h2, degt, b1.reshape(1, D_hid), lin_W,
        lin_b.reshape(1, D_out))
    return out


# speedup vs baseline: 1.0145x; 1.0102x over previous
import functools

import jax
import jax.numpy as jnp
from jax import lax
from jax.experimental import pallas as pl
from jax.experimental.pallas import tpu as pltpu
from jax.experimental.pallas import tpu_sc as plsc

NC = 2
NS = 16
L = 16
CHUNK = 64
STAGE = 16


def _round_up(v, m):
    return (v + m - 1) // m * m


def _mesh():
    return plsc.VectorSubcoreMesh(core_axis_name="c", subcore_axis_name="s")


@functools.lru_cache(maxsize=None)
def _make_deg(nch, NH):
    @functools.partial(
        pl.kernel,
        out_type=jax.ShapeDtypeStruct((NC * NS, NH), jnp.float32),
        mesh=_mesh(),
        compiler_params=pltpu.CompilerParams(needs_layout_passes=False),
        scratch_types=[
            pltpu.VMEM((nch, CHUNK), jnp.int32),
            pltpu.VMEM((NH,), jnp.float32),
        ],
    )
    def deg_k(dst_hbm, degp_hbm, didx, hist):
        c = lax.axis_index("c")
        s = lax.axis_index("s")
        w = s * NC + c

        def fill_z(k, carry):
            hist[pl.ds(k * L, L)] = jnp.zeros((L,), jnp.float32)
            return carry

        lax.fori_loop(0, NH // L, fill_z, 0)

        pltpu.sync_copy(dst_hbm.at[w], didx)
        ones = jnp.ones((L,), jnp.float32)

        def chunk_body(j, carry):
            def lane_grp(k, carry2):
                idx = didx[j, pl.ds(k * L, L)]
                plsc.addupdate_scatter(hist, [idx], ones)
                return carry2

            return lax.fori_loop(0, CHUNK // L, lane_grp, carry)

        lax.fori_loop(0, nch, chunk_body, 0)
        pltpu.sync_copy(hist, degp_hbm.at[w])

    return deg_k


FPT = 8
GR = 16
CH = 128


@functools.lru_cache(maxsize=None)
def _make_agg(N, D, nch, NP):
    @functools.partial(
        pl.kernel,
        out_type=jax.ShapeDtypeStruct((NC * NS, NP, FPT), jnp.float32),
        mesh=_mesh(),
        compiler_params=pltpu.CompilerParams(
            needs_layout_passes=False, use_tc_tiling_on_sc=False),
        scratch_types=[
            pltpu.VMEM((STAGE, CH), jnp.int32),
            pltpu.VMEM((STAGE, CH), jnp.int32),
            pltpu.VMEM((CH, GR), jnp.float32),
            pltpu.VMEM((CH, GR), jnp.float32),
            pltpu.VMEM((CH, GR), jnp.float32),
            pltpu.VMEM((CH, GR), jnp.float32),
            pltpu.VMEM((NP, FPT), jnp.float32),
            pltpu.SemaphoreType.DMA,
            pltpu.SemaphoreType.DMA,
            pltpu.SemaphoreType.DMA,
            pltpu.SemaphoreType.DMA,
        ],
    )
    def agg_k(src_hbm, dst_hbm, h2g_hbm, zrows_hbm, acc_hbm, sidx, didx,
              rows0, rows1, rows2, rows3, acc, sem0, sem1, sem2, sem3):
        c = lax.axis_index("c")
        s = lax.axis_index("s")
        t = s * NC + c
        half = t // 2
        off = (t % 2) * FPT

        pltpu.sync_copy(zrows_hbm, acc)

        iota = lax.iota(jnp.int32, L)
        voff = jnp.full((L,), off, jnp.int32)
        fl = [voff + f for f in range(FPT)]
        fa = [jnp.full((L,), f, jnp.int32) for f in range(FPT)]

        def do_adds(rows, j):
            @plsc.parallel_loop(0, CH // L, step=1, unroll=2)
            def grp_body(k):
                dst16 = didx[j, pl.ds(k * L, L)]
                eidx = iota + k * L
                for f in range(FPT):
                    vals = plsc.load_gather(rows, [eidx, fl[f]])
                    plsc.addupdate_scatter(acc, [dst16, fa[f]], vals)

        bufs = [(rows0, sem0), (rows1, sem1), (rows2, sem2), (rows3, sem3)]
        DEPTH = len(bufs)

        def gather(j, rows, sem):
            pltpu.async_copy(h2g_hbm.at[half].at[sidx.at[j]], rows, sem)

        def drain(j, rows, sem):
            pltpu.make_async_copy(h2g_hbm.at[half].at[sidx.at[j]], rows,
                                  sem).wait()

        def stage_body(g, carry):
            pltpu.sync_copy(src_hbm.at[pl.ds(g * STAGE, STAGE)], sidx)
            pltpu.sync_copy(dst_hbm.at[pl.ds(g * STAGE, STAGE)], didx)
            for j in range(DEPTH - 1):
                gather(j, *bufs[j])
            for j in range(STAGE):
                drain(j, *bufs[j % DEPTH])
                if j + DEPTH - 1 < STAGE:
                    gather(j + DEPTH - 1, *bufs[(j + DEPTH - 1) % DEPTH])
                do_adds(bufs[j % DEPTH][0], j)
            return carry

        lax.fori_loop(0, nch // STAGE, stage_body, 0)
        pltpu.sync_copy(acc, acc_hbm.at[t])

    return agg_k


def _tc1_body(x_ref, w_ref, d_ref, h2_ref):
    h = jnp.dot(x_ref[...], w_ref[...], preferred_element_type=jnp.float32)
    deg = jnp.sum(d_ref[...], axis=1, keepdims=True) + 1.0
    h2_ref[...] = h * lax.rsqrt(deg)


def _tc2_body(a_ref, h2_ref, d_ref, b1_ref, w_ref, lb_ref, o_ref):
    deg = jnp.sum(d_ref[...], axis=1, keepdims=True) + 1.0
    dis = lax.rsqrt(deg)
    t = jnp.tanh((a_ref[...] + h2_ref[...]) * dis + b1_ref[...])
    o_ref[...] = (jnp.dot(t, w_ref[...], preferred_element_type=jnp.float32)
                  + lb_ref[...])


@functools.lru_cache(maxsize=None)
def _make_tc1(N, D_in, D_hid, BR):
    grid = (N + BR - 1) // BR
    return pl.pallas_call(
        _tc1_body,
        grid=(grid,),
        in_specs=[
            pl.BlockSpec((BR, D_in), lambda i: (i, 0)),
            pl.BlockSpec((D_in, D_hid), lambda i: (0, 0)),
            pl.BlockSpec((BR, NC * NS), lambda i: (i, 0)),
        ],
        out_specs=pl.BlockSpec((BR, D_hid), lambda i: (i, 0)),
        out_shape=jax.ShapeDtypeStruct((N, D_hid), jnp.float32),
    )


@functools.lru_cache(maxsize=None)
def _make_tc2(N, D_hid, D_out, BR):
    grid = (N + BR - 1) // BR
    return pl.pallas_call(
        _tc2_body,
        grid=(grid,),
        in_specs=[
            pl.BlockSpec((BR, D_hid), lambda i: (i, 0)),
            pl.BlockSpec((BR, D_hid), lambda i: (i, 0)),
            pl.BlockSpec((BR, NC * NS), lambda i: (i, 0)),
            pl.BlockSpec((1, D_hid), lambda i: (0, 0)),
            pl.BlockSpec((D_hid, D_out), lambda i: (0, 0)),
            pl.BlockSpec((1, D_out), lambda i: (0, 0)),
        ],
        out_specs=pl.BlockSpec((BR, D_out), lambda i: (i, 0)),
        out_shape=jax.ShapeDtypeStruct((N, D_out), jnp.float32),
    )


def kernel(x, edge_index, W1, b1, lin_W, lin_b):
    N, D_in = x.shape
    D_hid = W1.shape[1]
    D_out = lin_W.shape[1]
    E = edge_index.shape[1]

    src = edge_index[0]
    dst = edge_index[1]

    EP = _round_up(E, max(NC * NS * CHUNK, STAGE * CH))
    pad = EP - E
    if pad:
        src = jnp.concatenate([src, jnp.zeros((pad,), src.dtype)])
        dst = jnp.concatenate([dst, jnp.full((pad,), N, dst.dtype)])

    nch_deg = EP // (NC * NS * CHUNK)
    nch_agg = EP // CH
    NH = _round_up(N + 1, 128)
    NP = _round_up(N + 1, 8)

    dst_deg = dst.reshape(NC * NS, nch_deg, CHUNK)
    src2 = src.reshape(nch_agg, CH)
    dst2 = dst.reshape(nch_agg, CH)

    degp = _make_deg(nch_deg, NH)(dst_deg)
    degt = degp[:, :N].T

    BR = 1000 if N % 1000 == 0 else N
    h2 = _make_tc1(N, D_in, D_hid, BR)(x, W1, degt)
    h2g = h2.reshape(N, D_hid // GR, GR).transpose(1, 0, 2)
    zrows = jnp.zeros((NP, FPT), jnp.float32)
    accg = _make_agg(N, D_hid, nch_agg, NP)(src2, dst2, h2g, zrows)
    acc = accg.transpose(1, 0, 2).reshape(NP, D_hid)
    out = _make_tc2(N, D_hid, D_out, BR)(
        acc, h2, degt, b1.reshape(1, D_hid), lin_W,
        lin_b.reshape(1, D_out))
    return out
